# baseline (speedup 1.0000x reference)
import functools

import jax
import jax.numpy as jnp
from jax.experimental import pallas as pl
from jax.experimental.pallas import tpu as pltpu
from jax.experimental.pallas import tpu_sc as plsc

_SC_CORES = 2
_SC_SUBCORES = 16
_SC_WORKERS = _SC_CORES * _SC_SUBCORES


def _sc_gather_elems(flat_tbl, gidx):
    nw, n = gidx.shape
    mesh = plsc.VectorSubcoreMesh(core_axis_name="c", subcore_axis_name="s")

    @pl.kernel(
        out_type=jax.ShapeDtypeStruct((nw, n), flat_tbl.dtype),
        mesh=mesh,
        scratch_types=[
            pltpu.VMEM((n,), jnp.int32),
            pltpu.VMEM((n,), flat_tbl.dtype),
            pltpu.SemaphoreType.DMA,
        ],
    )
    def gather_kernel(tbl_hbm, idx_hbm, out_hbm, idx_v, vals_v, sem):
        wid = jax.lax.axis_index("s") * _SC_CORES + jax.lax.axis_index("c")
        pltpu.sync_copy(idx_hbm.at[wid], idx_v)
        pltpu.async_copy(tbl_hbm.at[idx_v], vals_v, sem).wait()
        pltpu.sync_copy(vals_v, out_hbm.at[wid])

    return gather_kernel(flat_tbl, gidx)


def _pool_t_body(g_ref, mean_ref, *, batch, ctx):
    acc = g_ref[:, pl.ds(0, batch)]
    for j in range(1, ctx):
        acc = acc + g_ref[:, pl.ds(j * batch, batch)]
    mean_ref[...] = (acc * (1.0 / ctx)).astype(jnp.bfloat16)


def _pool_t(g_t, batch, ctx, interpret=False):
    d, n = g_t.shape
    return pl.pallas_call(
        functools.partial(_pool_t_body, batch=batch, ctx=ctx),
        in_specs=[pl.BlockSpec((d, n), lambda: (0, 0))],
        out_specs=pl.BlockSpec((d, batch), lambda: (0, 0)),
        out_shape=jax.ShapeDtypeStruct((d, batch), jnp.bfloat16),
        interpret=interpret,
    )(g_t)


def _mm_t_body(m_ref, w_ref, o_ref):
    o_ref[...] = jax.lax.dot_general(
        w_ref[...],
        m_ref[...],
        (((0,), (0,)), ((), ())),
        preferred_element_type=jnp.float32,
    )


def _project_t(mean_t, w_t, v_tile=4096, interpret=False):
    d, batch = mean_t.shape
    vocab = w_t.shape[1]
    grid = (pl.cdiv(vocab, v_tile),)
    return pl.pallas_call(
        _mm_t_body,
        grid=grid,
        in_specs=[
            pl.BlockSpec((d, batch), lambda i: (0, 0)),
            pl.BlockSpec((d, v_tile), lambda i: (0, i)),
        ],
        out_specs=pl.BlockSpec((v_tile, batch), lambda i: (i, 0)),
        out_shape=jax.ShapeDtypeStruct((vocab, batch), jnp.float32),
        compiler_params=pltpu.CompilerParams(dimension_semantics=("parallel",)),
        interpret=interpret,
    )(mean_t, w_t)


def kernel(contexts, emb_table, W_out):
    batch, ctx = contexts.shape
    vocab, d = emb_table.shape
    n = batch * ctx
    idx = contexts.T.reshape(n).astype(jnp.int32)
    flat_e = emb_table.T.reshape(d * vocab)
    gidx = jnp.arange(d, dtype=jnp.int32)[:, None] * vocab + idx[None, :]
    g_t = _sc_gather_elems(flat_e, gidx)
    mean_t = _pool_t(g_t, batch, ctx)
    w_t = W_out.astype(jnp.bfloat16).T
    return _project_t(mean_t, w_t).T


# speedup vs baseline: 1.1314x; 1.1314x over previous
import dataclasses
import functools

import jax
import jax.numpy as jnp
from jax.experimental import pallas as pl
from jax.experimental.pallas import tpu as pltpu
from jax.experimental.pallas import tpu_sc as plsc

_SC_CORES = 2
_SC_SUBCORES = 16
_SC_WORKERS = _SC_CORES * _SC_SUBCORES


def _sc_gather_t(tbl_t, idx):
    d, vocab = tbl_t.shape
    n = idx.shape[0]
    mesh = plsc.VectorSubcoreMesh(core_axis_name="c", subcore_axis_name="s")
    cp = pltpu.CompilerParams()
    if "needs_layout_passes" in pltpu.CompilerParams.__dataclass_fields__:
        cp = dataclasses.replace(cp, needs_layout_passes=False)

    @pl.kernel(
        out_type=jax.ShapeDtypeStruct((d, n), tbl_t.dtype),
        mesh=mesh,
        compiler_params=cp,
        scratch_types=[
            pltpu.VMEM((vocab,), tbl_t.dtype),
            pltpu.VMEM((n,), jnp.int32),
            pltpu.VMEM((n,), tbl_t.dtype),
            pltpu.SemaphoreType.DMA,
        ],
    )
    def gather_kernel(tbl_hbm, idx_hbm, out_hbm, row_v, idx_v, vals_v, sem):
        wid = jax.lax.axis_index("s") * _SC_CORES + jax.lax.axis_index("c")
        row_cp = pltpu.async_copy(tbl_hbm.at[wid], row_v, sem)
        pltpu.sync_copy(idx_hbm, idx_v)
        row_cp.wait()

        @pl.loop(0, n, step=16)
        def _(i):
            iv = idx_v[pl.ds(i, 16)]
            vals_v[pl.ds(i, 16)] = plsc.load_gather(row_v, [iv])

        pltpu.sync_copy(vals_v, out_hbm.at[wid])

    return gather_kernel(tbl_t, idx)


def _pool_t_body(g_ref, mean_ref, *, batch, ctx):
    acc = g_ref[:, pl.ds(0, batch)]
    for j in range(1, ctx):
        acc = acc + g_ref[:, pl.ds(j * batch, batch)]
    mean_ref[...] = (acc * (1.0 / ctx)).astype(jnp.bfloat16)


def _pool_t(g_t, batch, ctx, interpret=False):
    d, n = g_t.shape
    return pl.pallas_call(
        functools.partial(_pool_t_body, batch=batch, ctx=ctx),
        in_specs=[pl.BlockSpec((d, n), lambda: (0, 0))],
        out_specs=pl.BlockSpec((d, batch), lambda: (0, 0)),
        out_shape=jax.ShapeDtypeStruct((d, batch), jnp.bfloat16),
        interpret=interpret,
    )(g_t)


def _mm_t_body(m_ref, w_ref, o_ref):
    o_ref[...] = jax.lax.dot_general(
        w_ref[...],
        m_ref[...],
        (((0,), (0,)), ((), ())),
        preferred_element_type=jnp.float32,
    )


def _project_t(mean_t, w_t, v_tile=2048, interpret=False):
    d, batch = mean_t.shape
    vocab = w_t.shape[1]
    grid = (pl.cdiv(vocab, v_tile),)
    return pl.pallas_call(
        _mm_t_body,
        grid=grid,
        in_specs=[
            pl.BlockSpec((d, batch), lambda i: (0, 0)),
            pl.BlockSpec((d, v_tile), lambda i: (0, i)),
        ],
        out_specs=pl.BlockSpec((v_tile, batch), lambda i: (i, 0)),
        out_shape=jax.ShapeDtypeStruct((vocab, batch), jnp.float32),
        compiler_params=pltpu.CompilerParams(dimension_semantics=("parallel",)),
        interpret=interpret,
    )(mean_t, w_t)


def kernel(contexts, emb_table, W_out):
    batch, ctx = contexts.shape
    vocab, d = emb_table.shape
    n = batch * ctx
    idx = contexts.T.reshape(n).astype(jnp.int32)
    g_t = _sc_gather_t(emb_table.T, idx)
    mean_t = _pool_t(g_t, batch, ctx)
    w_t = W_out.astype(jnp.bfloat16).T
    return _project_t(mean_t, w_t).T


# speedup vs baseline: 1.1637x; 1.0286x over previous
import dataclasses
import functools

import jax
import jax.numpy as jnp
from jax.experimental import pallas as pl
from jax.experimental.pallas import tpu as pltpu
from jax.experimental.pallas import tpu_sc as plsc

_SC_CORES = 2
_SC_SUBCORES = 16
_SC_WORKERS = _SC_CORES * _SC_SUBCORES


def _sc_gather_pool_t(tbl_t, idx, batch, ctx):
    d, vocab = tbl_t.shape
    n = idx.shape[0]
    mesh = plsc.VectorSubcoreMesh(core_axis_name="c", subcore_axis_name="s")
    cp = pltpu.CompilerParams()
    if "needs_layout_passes" in pltpu.CompilerParams.__dataclass_fields__:
        cp = dataclasses.replace(cp, needs_layout_passes=False)

    @pl.kernel(
        out_type=jax.ShapeDtypeStruct((d, batch), tbl_t.dtype),
        mesh=mesh,
        compiler_params=cp,
        scratch_types=[
            pltpu.VMEM((vocab,), tbl_t.dtype),
            pltpu.VMEM((n,), jnp.int32),
            pltpu.VMEM((batch,), tbl_t.dtype),
            pltpu.SemaphoreType.DMA,
        ],
    )
    def gather_kernel(tbl_hbm, idx_hbm, out_hbm, row_v, idx_v, mean_v, sem):
        wid = jax.lax.axis_index("s") * _SC_CORES + jax.lax.axis_index("c")
        row_cp = pltpu.async_copy(tbl_hbm.at[wid], row_v, sem)
        pltpu.sync_copy(idx_hbm, idx_v)
        row_cp.wait()

        @pl.loop(0, batch, step=16)
        def _(b):
            acc = plsc.load_gather(row_v, [idx_v[pl.ds(b, 16)]])
            for j in range(1, ctx):
                acc = acc + plsc.load_gather(
                    row_v, [idx_v[pl.ds(j * batch + b, 16)]]
                )
            mean_v[pl.ds(b, 16)] = acc * (1.0 / ctx)

        pltpu.sync_copy(mean_v, out_hbm.at[wid])

    return gather_kernel(tbl_t, idx)


def _mm_t_body(m_ref, w_ref, o_ref):
    o_ref[...] = jax.lax.dot_general(
        w_ref[...],
        m_ref[...].astype(jnp.bfloat16),
        (((0,), (0,)), ((), ())),
        preferred_element_type=jnp.float32,
    )


def _project_t(mean_t, w_t, v_tile=2048, interpret=False):
    d, batch = mean_t.shape
    vocab = w_t.shape[1]
    grid = (pl.cdiv(vocab, v_tile),)
    return pl.pallas_call(
        _mm_t_body,
        grid=grid,
        in_specs=[
            pl.BlockSpec((d, batch), lambda i: (0, 0)),
            pl.BlockSpec((d, v_tile), lambda i: (0, i)),
        ],
        out_specs=pl.BlockSpec((v_tile, batch), lambda i: (i, 0)),
        out_shape=jax.ShapeDtypeStruct((vocab, batch), jnp.float32),
        compiler_params=pltpu.CompilerParams(dimension_semantics=("parallel",)),
        interpret=interpret,
    )(mean_t, w_t)


def kernel(contexts, emb_table, W_out):
    batch, ctx = contexts.shape
    vocab, d = emb_table.shape
    n = batch * ctx
    idx = contexts.T.reshape(n).astype(jnp.int32)
    mean_t = _sc_gather_pool_t(emb_table.T, idx, batch, ctx)
    w_t = W_out.astype(jnp.bfloat16).T
    return _project_t(mean_t, w_t).T
